# baseline (device time: 16239 ns/iter reference)
import jax
import jax.numpy as jnp
from jax import lax
from jax.experimental import pallas as pl
from jax.experimental.pallas import tpu as pltpu

N_GLOBAL = 2048.0
EPS = 1e-5


def kernel(x, gamma, beta):
    m, n_loc = x.shape

    def body(x_ref, g_ref, b_ref, out_ref, stats_ref, peer_ref, send_sem, recv_sem):
        my_x = lax.axis_index("x")
        my_y = lax.axis_index("y")
        peer = (my_x, 1 - my_y)

        barrier_sem = pltpu.get_barrier_semaphore()
        pl.semaphore_signal(
            barrier_sem, inc=1, device_id=peer,
            device_id_type=pl.DeviceIdType.MESH,
        )
        pl.semaphore_wait(barrier_sem, 1)

        xv = x_ref[:, :]
        stats_ref[0, :] = jnp.sum(xv, axis=1)
        stats_ref[1, :] = jnp.sum(xv * xv, axis=1)

        rdma = pltpu.make_async_remote_copy(
            src_ref=stats_ref,
            dst_ref=peer_ref,
            send_sem=send_sem,
            recv_sem=recv_sem,
            device_id=peer,
            device_id_type=pl.DeviceIdType.MESH,
        )
        rdma.start()
        rdma.wait()

        total_s = stats_ref[0, :] + peer_ref[0, :]
        total_sq = stats_ref[1, :] + peer_ref[1, :]
        mean = total_s / N_GLOBAL
        var = total_sq / N_GLOBAL - mean * mean
        inv = lax.rsqrt(var + EPS)
        norm = (xv - mean[:, None]) * inv[:, None]
        out_ref[:, :] = g_ref[0, :] * norm + b_ref[0, :]

    return pl.pallas_call(
        body,
        out_shape=jax.ShapeDtypeStruct((m, n_loc), jnp.float32),
        in_specs=[
            pl.BlockSpec(memory_space=pltpu.VMEM),
            pl.BlockSpec(memory_space=pltpu.VMEM),
            pl.BlockSpec(memory_space=pltpu.VMEM),
        ],
        out_specs=pl.BlockSpec(memory_space=pltpu.VMEM),
        scratch_shapes=[
            pltpu.VMEM((2, m), jnp.float32),
            pltpu.VMEM((2, m), jnp.float32),
            pltpu.SemaphoreType.DMA,
            pltpu.SemaphoreType.DMA,
        ],
        compiler_params=pltpu.CompilerParams(collective_id=0),
    )(x, gamma.reshape(1, n_loc), beta.reshape(1, n_loc))
